# Initial kernel scaffold; baseline (speedup 1.0000x reference)
import jax
import jax.numpy as jnp
from jax.experimental import pallas as pl


def kernel(hidden_states, gate_weight, gate_bias, expert_w_gate, expert_w_up, expert_w_down, shared_w_gate, shared_w_up, shared_w_down):
    raise NotImplementedError("write your pallas kernel here")


# speedup vs baseline: 1.5241x; 1.5241x over previous
import functools

import jax
import jax.numpy as jnp
from jax.experimental import pallas as pl
from jax.experimental.pallas import tpu as pltpu

NUM_EXPERTS = 8
TOP_K = 2
HIDDEN = 1024
INTER = 512
N_GROUP = 4
GSZ = NUM_EXPERTS // N_GROUP
TOPK_GROUP = 2
ROUTED_SCALE = 2.5
TOKENS = 2048

TBLK = 256

_INTERP = False


def _routing_body(x_ref, gwt_ref, gb_ref, scores_ref, mask_ref):
    x = x_ref[...]
    logits = jnp.dot(x, gwt_ref[...], preferred_element_type=jnp.float32)
    s = jax.nn.sigmoid(logits)
    swb = s + gb_ref[...]

    col = lambda a, i: a[:, i : i + 1]
    g = [sum(col(swb, gi * GSZ + j) for j in range(GSZ)) for gi in range(N_GROUP)]
    gsel = []
    for gi in range(N_GROUP):
        rank = jnp.zeros_like(g[gi])
        for gj in range(N_GROUP):
            if gj == gi:
                continue
            beats = g[gj] > g[gi]
            if gj < gi:
                beats = beats | (g[gj] == g[gi])
            rank = rank + beats.astype(jnp.float32)
        gsel.append(rank < TOPK_GROUP)
    swbm = [
        jnp.where(gsel[e // GSZ], col(swb, e), 0.0) for e in range(NUM_EXPERTS)
    ]
    sel = []
    for e in range(NUM_EXPERTS):
        rank = jnp.zeros_like(swbm[e])
        for e2 in range(NUM_EXPERTS):
            if e2 == e:
                continue
            beats = swbm[e2] > swbm[e]
            if e2 < e:
                beats = beats | (swbm[e2] == swbm[e])
            rank = rank + beats.astype(jnp.float32)
        sel.append(rank < TOP_K)
    sc = [jnp.where(sel[e], col(s, e), 0.0) for e in range(NUM_EXPERTS)]
    denom = sum(sc) + 1e-20
    w = [sc[e] / denom * ROUTED_SCALE for e in range(NUM_EXPERTS)]
    scores_ref[...] = jnp.concatenate(w, axis=1)
    mask_ref[...] = jnp.concatenate(
        [sel[e].astype(jnp.int32) for e in range(NUM_EXPERTS)], axis=1
    )


def _routing(x, gate_weight, gate_bias):
    nblk = TOKENS // TBLK
    return pl.pallas_call(
        _routing_body,
        grid=(nblk,),
        in_specs=[
            pl.BlockSpec((TBLK, HIDDEN), lambda t: (t, 0)),
            pl.BlockSpec((HIDDEN, NUM_EXPERTS), lambda t: (0, 0)),
            pl.BlockSpec((1, NUM_EXPERTS), lambda t: (0, 0)),
        ],
        out_specs=[
            pl.BlockSpec((TBLK, NUM_EXPERTS), lambda t: (t, 0)),
            pl.BlockSpec((TBLK, NUM_EXPERTS), lambda t: (t, 0)),
        ],
        out_shape=[
            jax.ShapeDtypeStruct((TOKENS, NUM_EXPERTS), jnp.float32),
            jax.ShapeDtypeStruct((TOKENS, NUM_EXPERTS), jnp.int32),
        ],
        interpret=_INTERP,
    )(x, gate_weight.T, gate_bias.reshape(1, NUM_EXPERTS))


def _shared_body(x_ref, wg_ref, wu_ref, wd_ref, out_ref):
    x = x_ref[...]
    hg = jnp.dot(x, wg_ref[...], preferred_element_type=jnp.float32)
    hu = jnp.dot(x, wu_ref[...], preferred_element_type=jnp.float32)
    h = jax.nn.silu(hg) * hu
    out_ref[...] = jnp.dot(h, wd_ref[...], preferred_element_type=jnp.float32)


def _shared(x, wg, wu, wd):
    nblk = TOKENS // TBLK
    return pl.pallas_call(
        _shared_body,
        grid=(nblk,),
        in_specs=[
            pl.BlockSpec((TBLK, HIDDEN), lambda t: (t, 0)),
            pl.BlockSpec((HIDDEN, INTER), lambda t: (0, 0)),
            pl.BlockSpec((HIDDEN, INTER), lambda t: (0, 0)),
            pl.BlockSpec((INTER, HIDDEN), lambda t: (0, 0)),
        ],
        out_specs=pl.BlockSpec((TBLK, HIDDEN), lambda t: (t, 0)),
        out_shape=jax.ShapeDtypeStruct((TOKENS, HIDDEN), jnp.float32),
        interpret=_INTERP,
    )(x, wg, wu, wd)


def _dense_body(x_ref, wg_ref, wu_ref, wd_ref, sc_ref, shared_ref, out_ref):
    e = pl.program_id(0)
    x = x_ref[...]
    hg = jnp.dot(x, wg_ref[0], preferred_element_type=jnp.float32)
    hu = jnp.dot(x, wu_ref[0], preferred_element_type=jnp.float32)
    h = jax.nn.silu(hg) * hu
    out_e = jnp.dot(h, wd_ref[0], preferred_element_type=jnp.float32)
    onehot = (
        jax.lax.broadcasted_iota(jnp.int32, (NUM_EXPERTS, 1), 0) == e
    ).astype(jnp.float32)
    w_col = jnp.dot(sc_ref[...], onehot, preferred_element_type=jnp.float32)

    @pl.when(e == 0)
    def _():
        out_ref[...] = shared_ref[...] + w_col * out_e

    @pl.when(e > 0)
    def _():
        out_ref[...] = out_ref[...] + w_col * out_e


def _dense_moe(x, ewg, ewu, ewd, scores, shared_out):
    return pl.pallas_call(
        _dense_body,
        grid=(NUM_EXPERTS,),
        in_specs=[
            pl.BlockSpec((TOKENS, HIDDEN), lambda e: (0, 0)),
            pl.BlockSpec((1, HIDDEN, INTER), lambda e: (e, 0, 0)),
            pl.BlockSpec((1, HIDDEN, INTER), lambda e: (e, 0, 0)),
            pl.BlockSpec((1, INTER, HIDDEN), lambda e: (e, 0, 0)),
            pl.BlockSpec((TOKENS, NUM_EXPERTS), lambda e: (0, 0)),
            pl.BlockSpec((TOKENS, HIDDEN), lambda e: (0, 0)),
        ],
        out_specs=pl.BlockSpec((TOKENS, HIDDEN), lambda e: (0, 0)),
        out_shape=jax.ShapeDtypeStruct((TOKENS, HIDDEN), jnp.float32),
        interpret=_INTERP,
    )(x, ewg, ewu, ewd, scores, shared_out)


def kernel(hidden_states, gate_weight, gate_bias, expert_w_gate, expert_w_up,
           expert_w_down, shared_w_gate, shared_w_up, shared_w_down):
    x = hidden_states.astype(jnp.float32)
    scores, _mask = _routing(x, gate_weight, gate_bias)
    shared_out = _shared(x, shared_w_gate, shared_w_up, shared_w_down)
    return _dense_moe(x, expert_w_gate, expert_w_up, expert_w_down, scores,
                      shared_out)
